# baseline (speedup 1.0000x reference)
import functools

import jax
import jax.numpy as jnp
from jax import lax
from jax.experimental import pallas as pl
from jax.experimental.pallas import tpu as pltpu
from jax.experimental.pallas import tpu_sc as plsc

_CAPACITY_FACTOR = 0.5
_AUX_LOSS_WEIGHT = 0.01

_RT = 256


def _ranks_body(nc, row_ref, chunk_ref, out_ref):
    n = row_ref.shape[-1]
    g = pl.program_id(0)
    c = g % nc
    row = row_ref[0, 0, :].reshape(1, n)
    chunk = chunk_ref[0, 0, :].reshape(_RT, 1)
    col_i = lax.broadcasted_iota(jnp.int32, (_RT, n), 1)
    row_i = lax.broadcasted_iota(jnp.int32, (_RT, n), 0) + c * _RT
    gt = row > chunk
    eq = row == chunk
    tie = col_i < row_i
    contrib = jnp.where(gt | (eq & tie), 1, 0)
    out_ref[0, 0, :] = jnp.sum(contrib, axis=1)


def _scatter_body(nk, row_ref, rank_ref, tks_ref, tki_ref, flat_ref):
    n = row_ref.shape[-1]
    h = pl.program_id(0)
    bi = h // nk
    rc = h % nk
    row = row_ref[0, 0, :].reshape(1, n)
    ranks = rank_ref[0, 0, :].reshape(1, n)
    col_i = lax.broadcasted_iota(jnp.int32, (_RT, n), 1)
    riota = lax.broadcasted_iota(jnp.int32, (_RT, n), 0) + rc * _RT
    mask = ranks == riota
    tks = jnp.sum(jnp.where(mask, row, 0.0), axis=1)
    tki = jnp.sum(jnp.where(mask, col_i, 0), axis=1)
    tks_ref[0, 0, :] = tks
    tki_ref[0, 0, :] = tki
    flat_ref[0, 0, :] = tki + bi * n


def _aux_body(s_ref, aux_ref):
    b, n = s_ref.shape
    p = jnp.sum(s_ref[...], axis=1) / n
    aux = _AUX_LOSS_WEIGHT * jnp.mean((p - _CAPACITY_FACTOR) ** 2)
    aux_ref[...] = aux.reshape(1, 1)


def _topk(scores):
    b, n = scores.shape
    k = n // 2
    nc = n // _RT
    nk = k // _RT
    scores3 = scores.reshape(b, 1, n)
    chunks = scores.reshape(b * nc, 1, _RT)

    ranks = pl.pallas_call(
        functools.partial(_ranks_body, nc),
        grid=(b * nc,),
        in_specs=[
            pl.BlockSpec((1, 1, n), lambda g: (g // nc, 0, 0)),
            pl.BlockSpec((1, 1, _RT), lambda g: (g, 0, 0)),
        ],
        out_specs=pl.BlockSpec((1, 1, _RT), lambda g: (g, 0, 0)),
        out_shape=jax.ShapeDtypeStruct((b * nc, 1, _RT), jnp.int32),
    )(scores3, chunks)

    tks, tki, flat = pl.pallas_call(
        functools.partial(_scatter_body, nk),
        grid=(b * nk,),
        in_specs=[
            pl.BlockSpec((1, 1, n), lambda h: (h // nk, 0, 0)),
            pl.BlockSpec((1, 1, n), lambda h: (h // nk, 0, 0)),
        ],
        out_specs=[
            pl.BlockSpec((1, 1, _RT), lambda h: (h, 0, 0)),
            pl.BlockSpec((1, 1, _RT), lambda h: (h, 0, 0)),
            pl.BlockSpec((1, 1, _RT), lambda h: (h, 0, 0)),
        ],
        out_shape=(
            jax.ShapeDtypeStruct((b * nk, 1, _RT), jnp.float32),
            jax.ShapeDtypeStruct((b * nk, 1, _RT), jnp.int32),
            jax.ShapeDtypeStruct((b * nk, 1, _RT), jnp.int32),
        ),
    )(scores3, ranks.reshape(b, 1, n))

    aux = pl.pallas_call(
        _aux_body,
        out_shape=jax.ShapeDtypeStruct((1, 1), jnp.float32),
    )(scores)

    return (tks.reshape(b, k), tki.reshape(b, k), flat.reshape(b * k),
            aux.reshape(()))


def _gather_rows(hs_flat, flat_idx):
    rows, d = hs_flat.shape
    n_sel = flat_idx.shape[0]
    info = plsc.get_sparse_core_info()
    nw = info.num_cores * info.num_subcores
    per_w = n_sel // nw
    ch = 8
    n_ch = per_w // ch
    mesh = plsc.VectorSubcoreMesh(core_axis_name="c", subcore_axis_name="s")

    @functools.partial(
        pl.kernel, mesh=mesh,
        out_type=jax.ShapeDtypeStruct((n_sel, d), jnp.float32),
        scratch_types=[
            pltpu.VMEM((ch,), jnp.int32),
            pltpu.VMEM((ch, d), jnp.float32),
            pltpu.SemaphoreType.DMA,
        ],
    )
    def k(hs_hbm, idx_hbm, out_hbm, idx_v, rows_v, sem):
        wid = lax.axis_index("s") * info.num_cores + lax.axis_index("c")
        base = wid * per_w

        def body(c, carry):
            start = base + c * ch
            pltpu.sync_copy(idx_hbm.at[pl.ds(start, ch)], idx_v)
            pltpu.async_copy(hs_hbm.at[idx_v], rows_v, sem).wait()
            pltpu.sync_copy(rows_v, out_hbm.at[pl.ds(start, ch)])
            return carry

        lax.fori_loop(0, n_ch, body, 0)

    return k(hs_flat, flat_idx)


def kernel(hidden_states, router_weight, router_bias):
    b, s, d = hidden_states.shape
    k = min(int(s * _CAPACITY_FACTOR), s)
    scores = jnp.einsum('bsd,d->bs', hidden_states, router_weight) + router_bias
    scores = jax.nn.sigmoid(scores)
    topk_scores, topk_indices, flat_idx, aux = _topk(scores)
    selected = _gather_rows(
        hidden_states.reshape(b * s, d), flat_idx
    ).reshape(b, k, d)
    return (selected, topk_indices, topk_scores, aux)


# speedup vs baseline: 1.1059x; 1.1059x over previous
import functools

import jax
import jax.numpy as jnp
from jax import lax
from jax.experimental import pallas as pl
from jax.experimental.pallas import tpu as pltpu
from jax.experimental.pallas import tpu_sc as plsc

_CAPACITY_FACTOR = 0.5
_AUX_LOSS_WEIGHT = 0.01

_RT = 256


def _ranks_body(nc, row_ref, chunk_ref, out_ref):
    n = row_ref.shape[-1]
    g = pl.program_id(0)
    c = g % nc
    row = row_ref[0, 0, :].reshape(1, n)
    chunk = chunk_ref[0, 0, :].reshape(_RT, 1)
    col_i = lax.broadcasted_iota(jnp.int32, (_RT, n), 1)
    row_i = lax.broadcasted_iota(jnp.int32, (_RT, n), 0) + c * _RT
    gt = row > chunk
    eq = row == chunk
    tie = col_i < row_i
    contrib = jnp.where(gt | (eq & tie), 1, 0)
    out_ref[0, 0, :] = jnp.sum(contrib, axis=1)


def _scatter_body(nk, row_ref, rank_ref, tks_ref, tki_ref, flat_ref):
    n = row_ref.shape[-1]
    h = pl.program_id(0)
    bi = h // nk
    rc = h % nk
    row = row_ref[0, 0, :].reshape(1, n)
    ranks = rank_ref[0, 0, :].reshape(1, n)
    col_i = lax.broadcasted_iota(jnp.int32, (_RT, n), 1)
    riota = lax.broadcasted_iota(jnp.int32, (_RT, n), 0) + rc * _RT
    mask = ranks == riota
    tks = jnp.sum(jnp.where(mask, row, 0.0), axis=1)
    tki = jnp.sum(jnp.where(mask, col_i, 0), axis=1)
    tks_ref[0, 0, :] = tks
    tki_ref[0, 0, :] = tki
    flat_ref[0, 0, :] = tki + bi * n


def _aux_body(s_ref, aux_ref):
    b, n = s_ref.shape
    p = jnp.sum(s_ref[...], axis=1) / n
    aux = _AUX_LOSS_WEIGHT * jnp.mean((p - _CAPACITY_FACTOR) ** 2)
    aux_ref[...] = aux.reshape(1, 1)


def _topk(scores):
    b, n = scores.shape
    k = n // 2
    nc = n // _RT
    nk = k // _RT
    scores3 = scores.reshape(b, 1, n)
    chunks = scores.reshape(b * nc, 1, _RT)

    ranks = pl.pallas_call(
        functools.partial(_ranks_body, nc),
        grid=(b * nc,),
        in_specs=[
            pl.BlockSpec((1, 1, n), lambda g: (g // nc, 0, 0)),
            pl.BlockSpec((1, 1, _RT), lambda g: (g, 0, 0)),
        ],
        out_specs=pl.BlockSpec((1, 1, _RT), lambda g: (g, 0, 0)),
        out_shape=jax.ShapeDtypeStruct((b * nc, 1, _RT), jnp.int32),
    )(scores3, chunks)

    tks, tki, flat = pl.pallas_call(
        functools.partial(_scatter_body, nk),
        grid=(b * nk,),
        in_specs=[
            pl.BlockSpec((1, 1, n), lambda h: (h // nk, 0, 0)),
            pl.BlockSpec((1, 1, n), lambda h: (h // nk, 0, 0)),
        ],
        out_specs=[
            pl.BlockSpec((1, 1, _RT), lambda h: (h, 0, 0)),
            pl.BlockSpec((1, 1, _RT), lambda h: (h, 0, 0)),
            pl.BlockSpec((1, 1, _RT), lambda h: (h, 0, 0)),
        ],
        out_shape=(
            jax.ShapeDtypeStruct((b * nk, 1, _RT), jnp.float32),
            jax.ShapeDtypeStruct((b * nk, 1, _RT), jnp.int32),
            jax.ShapeDtypeStruct((b * nk, 1, _RT), jnp.int32),
        ),
    )(scores3, ranks.reshape(b, 1, n))

    aux = pl.pallas_call(
        _aux_body,
        out_shape=jax.ShapeDtypeStruct((1, 1), jnp.float32),
    )(scores)

    return (tks.reshape(b, k), tki.reshape(b, k), flat.reshape(b * k),
            aux.reshape(()))


def _gather_rows(hs_flat, flat_idx):
    rows, d = hs_flat.shape
    n_sel = flat_idx.shape[0]
    info = plsc.get_sparse_core_info()
    nw = info.num_cores * info.num_subcores
    per_w = n_sel // nw
    ch = 8
    n_ch = per_w // ch
    mesh = plsc.VectorSubcoreMesh(core_axis_name="c", subcore_axis_name="s")

    @functools.partial(
        pl.kernel, mesh=mesh,
        out_type=jax.ShapeDtypeStruct((n_sel, d), jnp.float32),
        scratch_types=[
            pltpu.VMEM((per_w,), jnp.int32),
            pltpu.VMEM((ch, d), jnp.float32),
            pltpu.VMEM((ch, d), jnp.float32),
            pltpu.SemaphoreType.DMA,
            pltpu.SemaphoreType.DMA,
        ],
    )
    def k(hs_hbm, idx_hbm, out_hbm, idx_v, rows0, rows1, sem0, sem1):
        wid = lax.axis_index("s") * info.num_cores + lax.axis_index("c")
        base = wid * per_w
        npairs = n_ch // 2

        pltpu.sync_copy(idx_hbm.at[pl.ds(base, per_w)], idx_v)

        def gather(c, buf, sem):
            src = hs_hbm.at[idx_v.at[pl.ds(c * ch, ch)]]
            pltpu.async_copy(src, buf, sem)

        def wait(buf, sem):
            pltpu.make_async_copy(hs_hbm.at[pl.ds(0, ch)], buf, sem).wait()

        def drain(c, buf, sem):
            wait(buf, sem)
            pltpu.sync_copy(buf, out_hbm.at[pl.ds(base + c * ch, ch)])

        gather(0, rows0, sem0)

        def body(t, carry):
            c0 = 2 * t
            gather(c0 + 1, rows1, sem1)
            drain(c0, rows0, sem0)
            gather(c0 + 2, rows0, sem0)
            drain(c0 + 1, rows1, sem1)
            return carry

        lax.fori_loop(0, npairs - 1, body, 0)
        c0 = 2 * (npairs - 1)
        gather(c0 + 1, rows1, sem1)
        drain(c0, rows0, sem0)
        drain(c0 + 1, rows1, sem1)

    return k(hs_flat, flat_idx)


def kernel(hidden_states, router_weight, router_bias):
    b, s, d = hidden_states.shape
    k = min(int(s * _CAPACITY_FACTOR), s)
    scores = jnp.einsum('bsd,d->bs', hidden_states, router_weight) + router_bias
    scores = jax.nn.sigmoid(scores)
    topk_scores, topk_indices, flat_idx, aux = _topk(scores)
    selected = _gather_rows(
        hidden_states.reshape(b * s, d), flat_idx
    ).reshape(b, k, d)
    return (selected, topk_indices, topk_scores, aux)


# speedup vs baseline: 1.6266x; 1.4709x over previous
import functools

import jax
import jax.numpy as jnp
from jax import lax
from jax.experimental import pallas as pl
from jax.experimental.pallas import tpu as pltpu
from jax.experimental.pallas import tpu_sc as plsc

_CAPACITY_FACTOR = 0.5
_AUX_LOSS_WEIGHT = 0.01

_RT = 256


def _roll(x, shift, axis):
    if shift < 0:
        shift += x.shape[axis]
    return pltpu.roll(x, shift, axis)


def _sort_body(n, s_ref, tks_ref, tki_ref, flat_ref):
    rows = n // 128
    bi = pl.program_id(0)
    v = s_ref[0, 0, :].reshape(rows, 128)
    gi = (lax.broadcasted_iota(jnp.int32, (rows, 128), 0) * 128
          + lax.broadcasted_iota(jnp.int32, (rows, 128), 1))
    idx = gi
    logn = n.bit_length() - 1
    for k in range(1, logn + 1):
        dirbit = ((gi >> k) & 1) == 1
        for j in range(k - 1, -1, -1):
            d = 1 << j
            if d < 128:
                axis, sh = 1, d
            else:
                axis, sh = 0, d // 128
            bit0 = (gi & d) == 0
            pv = jnp.where(bit0, _roll(v, -sh, axis), _roll(v, sh, axis))
            pi = jnp.where(bit0, _roll(idx, -sh, axis), _roll(idx, sh, axis))
            pv_before = (pv > v) | ((pv == v) & (pi < idx))
            flip = (~bit0) != dirbit
            swap = pv_before != flip
            v = jnp.where(swap, pv, v)
            idx = jnp.where(swap, pi, idx)
    kk = n // 2
    krows = kk // 128
    tks_ref[0, 0, :] = v[:krows, :].reshape(kk)
    tki_ref[0, 0, :] = idx[:krows, :].reshape(kk)
    flat_ref[0, 0, :] = idx[:krows, :].reshape(kk) + bi * n


def _ranks_body(nc, row_ref, chunk_ref, out_ref):
    n = row_ref.shape[-1]
    g = pl.program_id(0)
    c = g % nc
    row = row_ref[0, 0, :].reshape(1, n)
    chunk = chunk_ref[0, 0, :].reshape(_RT, 1)
    col_i = lax.broadcasted_iota(jnp.int32, (_RT, n), 1)
    row_i = lax.broadcasted_iota(jnp.int32, (_RT, n), 0) + c * _RT
    gt = row > chunk
    eq = row == chunk
    tie = col_i < row_i
    contrib = jnp.where(gt | (eq & tie), 1, 0)
    out_ref[0, 0, :] = jnp.sum(contrib, axis=1)


def _scatter_body(nk, row_ref, rank_ref, tks_ref, tki_ref, flat_ref):
    n = row_ref.shape[-1]
    h = pl.program_id(0)
    bi = h // nk
    rc = h % nk
    row = row_ref[0, 0, :].reshape(1, n)
    ranks = rank_ref[0, 0, :].reshape(1, n)
    col_i = lax.broadcasted_iota(jnp.int32, (_RT, n), 1)
    riota = lax.broadcasted_iota(jnp.int32, (_RT, n), 0) + rc * _RT
    mask = ranks == riota
    tks = jnp.sum(jnp.where(mask, row, 0.0), axis=1)
    tki = jnp.sum(jnp.where(mask, col_i, 0), axis=1)
    tks_ref[0, 0, :] = tks
    tki_ref[0, 0, :] = tki
    flat_ref[0, 0, :] = tki + bi * n


def _aux_body(s_ref, aux_ref):
    b, n = s_ref.shape
    p = jnp.sum(s_ref[...], axis=1) / n
    aux = _AUX_LOSS_WEIGHT * jnp.mean((p - _CAPACITY_FACTOR) ** 2)
    aux_ref[...] = aux.reshape(1, 1)


def _topk(scores):
    b, n = scores.shape
    k = n // 2
    scores3 = scores.reshape(b, 1, n)

    tks, tki, flat = pl.pallas_call(
        functools.partial(_sort_body, n),
        grid=(b,),
        in_specs=[
            pl.BlockSpec((1, 1, n), lambda g: (g, 0, 0)),
        ],
        out_specs=[
            pl.BlockSpec((1, 1, k), lambda g: (g, 0, 0)),
            pl.BlockSpec((1, 1, k), lambda g: (g, 0, 0)),
            pl.BlockSpec((1, 1, k), lambda g: (g, 0, 0)),
        ],
        out_shape=(
            jax.ShapeDtypeStruct((b, 1, k), jnp.float32),
            jax.ShapeDtypeStruct((b, 1, k), jnp.int32),
            jax.ShapeDtypeStruct((b, 1, k), jnp.int32),
        ),
    )(scores3)

    aux = pl.pallas_call(
        _aux_body,
        out_shape=jax.ShapeDtypeStruct((1, 1), jnp.float32),
    )(scores)

    return (tks.reshape(b, k), tki.reshape(b, k), flat.reshape(b * k),
            aux.reshape(()))


def _gather_rows(hs_flat, flat_idx):
    rows, d = hs_flat.shape
    n_sel = flat_idx.shape[0]
    info = plsc.get_sparse_core_info()
    nw = info.num_cores * info.num_subcores
    per_w = n_sel // nw
    ch = 8
    n_ch = per_w // ch
    mesh = plsc.VectorSubcoreMesh(core_axis_name="c", subcore_axis_name="s")

    @functools.partial(
        pl.kernel, mesh=mesh,
        out_type=jax.ShapeDtypeStruct((n_sel, d), jnp.float32),
        scratch_types=[
            pltpu.VMEM((per_w,), jnp.int32),
            pltpu.VMEM((ch, d), jnp.float32),
            pltpu.VMEM((ch, d), jnp.float32),
            pltpu.SemaphoreType.DMA,
            pltpu.SemaphoreType.DMA,
        ],
    )
    def k(hs_hbm, idx_hbm, out_hbm, idx_v, rows0, rows1, sem0, sem1):
        wid = lax.axis_index("s") * info.num_cores + lax.axis_index("c")
        base = wid * per_w
        npairs = n_ch // 2

        pltpu.sync_copy(idx_hbm.at[pl.ds(base, per_w)], idx_v)

        def gather(c, buf, sem):
            src = hs_hbm.at[idx_v.at[pl.ds(c * ch, ch)]]
            pltpu.async_copy(src, buf, sem)

        def wait(buf, sem):
            pltpu.make_async_copy(hs_hbm.at[pl.ds(0, ch)], buf, sem).wait()

        def drain(c, buf, sem):
            wait(buf, sem)
            pltpu.sync_copy(buf, out_hbm.at[pl.ds(base + c * ch, ch)])

        gather(0, rows0, sem0)

        def body(t, carry):
            c0 = 2 * t
            gather(c0 + 1, rows1, sem1)
            drain(c0, rows0, sem0)
            gather(c0 + 2, rows0, sem0)
            drain(c0 + 1, rows1, sem1)
            return carry

        lax.fori_loop(0, npairs - 1, body, 0)
        c0 = 2 * (npairs - 1)
        gather(c0 + 1, rows1, sem1)
        drain(c0, rows0, sem0)
        drain(c0 + 1, rows1, sem1)

    return k(hs_flat, flat_idx)


def kernel(hidden_states, router_weight, router_bias):
    b, s, d = hidden_states.shape
    k = min(int(s * _CAPACITY_FACTOR), s)
    scores = jnp.einsum('bsd,d->bs', hidden_states, router_weight) + router_bias
    scores = jax.nn.sigmoid(scores)
    topk_scores, topk_indices, flat_idx, aux = _topk(scores)
    selected = _gather_rows(
        hidden_states.reshape(b * s, d), flat_idx
    ).reshape(b, k, d)
    return (selected, topk_indices, topk_scores, aux)


# speedup vs baseline: 1.6296x; 1.0018x over previous
import functools

import jax
import jax.numpy as jnp
from jax import lax
from jax.experimental import pallas as pl
from jax.experimental.pallas import tpu as pltpu
from jax.experimental.pallas import tpu_sc as plsc

_CAPACITY_FACTOR = 0.5
_AUX_LOSS_WEIGHT = 0.01

_RT = 256


def _roll(x, shift, axis):
    if shift < 0:
        shift += x.shape[axis]
    return pltpu.roll(x, shift, axis)


def _sort_body(n, s_ref, tks_ref, tki_ref, flat_ref):
    rows = n // 128
    bi = pl.program_id(0)
    v = s_ref[0, 0, :].reshape(rows, 128)
    gi = (lax.broadcasted_iota(jnp.int32, (rows, 128), 0) * 128
          + lax.broadcasted_iota(jnp.int32, (rows, 128), 1))
    idx = gi
    logn = n.bit_length() - 1
    for k in range(1, logn + 1):
        dirbit = ((gi >> k) & 1) == 1
        for j in range(k - 1, -1, -1):
            d = 1 << j
            if d < 128:
                axis, sh = 1, d
            else:
                axis, sh = 0, d // 128
            bit0 = (gi & d) == 0
            pv = jnp.where(bit0, _roll(v, -sh, axis), _roll(v, sh, axis))
            pi = jnp.where(bit0, _roll(idx, -sh, axis), _roll(idx, sh, axis))
            pv_before = (pv > v) | ((pv == v) & (pi < idx))
            flip = (~bit0) != dirbit
            swap = pv_before != flip
            v = jnp.where(swap, pv, v)
            idx = jnp.where(swap, pi, idx)
    kk = n // 2
    krows = kk // 128
    tks_ref[0, 0, :] = v[:krows, :].reshape(kk)
    tki_ref[0, 0, :] = idx[:krows, :].reshape(kk)
    flat_ref[0, 0, :] = idx[:krows, :].reshape(kk) + bi * n


def _ranks_body(nc, row_ref, chunk_ref, out_ref):
    n = row_ref.shape[-1]
    g = pl.program_id(0)
    c = g % nc
    row = row_ref[0, 0, :].reshape(1, n)
    chunk = chunk_ref[0, 0, :].reshape(_RT, 1)
    col_i = lax.broadcasted_iota(jnp.int32, (_RT, n), 1)
    row_i = lax.broadcasted_iota(jnp.int32, (_RT, n), 0) + c * _RT
    gt = row > chunk
    eq = row == chunk
    tie = col_i < row_i
    contrib = jnp.where(gt | (eq & tie), 1, 0)
    out_ref[0, 0, :] = jnp.sum(contrib, axis=1)


def _scatter_body(nk, row_ref, rank_ref, tks_ref, tki_ref, flat_ref):
    n = row_ref.shape[-1]
    h = pl.program_id(0)
    bi = h // nk
    rc = h % nk
    row = row_ref[0, 0, :].reshape(1, n)
    ranks = rank_ref[0, 0, :].reshape(1, n)
    col_i = lax.broadcasted_iota(jnp.int32, (_RT, n), 1)
    riota = lax.broadcasted_iota(jnp.int32, (_RT, n), 0) + rc * _RT
    mask = ranks == riota
    tks = jnp.sum(jnp.where(mask, row, 0.0), axis=1)
    tki = jnp.sum(jnp.where(mask, col_i, 0), axis=1)
    tks_ref[0, 0, :] = tks
    tki_ref[0, 0, :] = tki
    flat_ref[0, 0, :] = tki + bi * n


def _aux_body(s_ref, aux_ref):
    b, n = s_ref.shape
    p = jnp.sum(s_ref[...], axis=1) / n
    aux = _AUX_LOSS_WEIGHT * jnp.mean((p - _CAPACITY_FACTOR) ** 2)
    aux_ref[...] = aux.reshape(1, 1)


def _topk(scores):
    b, n = scores.shape
    k = n // 2
    scores3 = scores.reshape(b, 1, n)

    tks, tki, flat = pl.pallas_call(
        functools.partial(_sort_body, n),
        grid=(b,),
        in_specs=[
            pl.BlockSpec((1, 1, n), lambda g: (g, 0, 0)),
        ],
        out_specs=[
            pl.BlockSpec((1, 1, k), lambda g: (g, 0, 0)),
            pl.BlockSpec((1, 1, k), lambda g: (g, 0, 0)),
            pl.BlockSpec((1, 1, k), lambda g: (g, 0, 0)),
        ],
        out_shape=(
            jax.ShapeDtypeStruct((b, 1, k), jnp.float32),
            jax.ShapeDtypeStruct((b, 1, k), jnp.int32),
            jax.ShapeDtypeStruct((b, 1, k), jnp.int32),
        ),
    )(scores3)

    aux = pl.pallas_call(
        _aux_body,
        out_shape=jax.ShapeDtypeStruct((1, 1), jnp.float32),
    )(scores)

    return (tks.reshape(b, k), tki.reshape(b, k), flat.reshape(b * k),
            aux.reshape(()))


def _gather_rows(hs_flat, flat_idx):
    rows, d = hs_flat.shape
    n_sel = flat_idx.shape[0]
    info = plsc.get_sparse_core_info()
    nw = info.num_cores * info.num_subcores
    per_w = n_sel // nw
    ch = 8
    n_ch = per_w // ch
    mesh = plsc.VectorSubcoreMesh(core_axis_name="c", subcore_axis_name="s")

    assert n_ch % 3 == 2, n_ch

    @functools.partial(
        pl.kernel, mesh=mesh,
        out_type=jax.ShapeDtypeStruct((n_sel, d), jnp.float32),
        scratch_types=[
            pltpu.VMEM((per_w,), jnp.int32),
            pltpu.VMEM((ch, d), jnp.float32),
            pltpu.VMEM((ch, d), jnp.float32),
            pltpu.VMEM((ch, d), jnp.float32),
            pltpu.SemaphoreType.DMA,
            pltpu.SemaphoreType.DMA,
            pltpu.SemaphoreType.DMA,
            pltpu.SemaphoreType.DMA,
            pltpu.SemaphoreType.DMA,
            pltpu.SemaphoreType.DMA,
        ],
    )
    def k(hs_hbm, idx_hbm, out_hbm, idx_v, r0, r1, r2, g0, g1, g2, w0, w1, w2):
        wid = lax.axis_index("s") * info.num_cores + lax.axis_index("c")
        base = wid * per_w
        bufs = (r0, r1, r2)
        gsems = (g0, g1, g2)
        wsems = (w0, w1, w2)

        pltpu.sync_copy(idx_hbm.at[pl.ds(base, per_w)], idx_v)

        def gather(c, u):
            src = hs_hbm.at[idx_v.at[pl.ds(c * ch, ch)]]
            pltpu.async_copy(src, bufs[u], gsems[u])

        def wait_gather(u):
            pltpu.make_async_copy(hs_hbm.at[pl.ds(0, ch)], bufs[u], gsems[u]).wait()

        def start_write(c, u):
            pltpu.async_copy(bufs[u], out_hbm.at[pl.ds(base + c * ch, ch)],
                             wsems[u])

        def wait_write(u):
            pltpu.make_async_copy(bufs[u], out_hbm.at[pl.ds(base, ch)],
                                  wsems[u]).wait()

        gather(0, 0)
        gather(1, 1)

        def body(t, carry):
            c0 = 3 * t
            for u in range(3):
                c = c0 + u
                v = (u + 2) % 3
                wait_gather(u)
                start_write(c, u)
                if u == 0:
                    @pl.when(t > 0)
                    def _():
                        wait_write(v)
                else:
                    wait_write(v)
                gather(c + 2, v)
            return carry

        lax.fori_loop(0, n_ch // 3, body, 0)
        c0 = (n_ch // 3) * 3
        for u, c in enumerate(range(c0, n_ch)):
            wait_gather(u)
            start_write(c, u)
        wait_write(2)
        for u in range(n_ch - c0):
            wait_write(u)

    return k(hs_flat, flat_idx)


def kernel(hidden_states, router_weight, router_bias):
    b, s, d = hidden_states.shape
    k = min(int(s * _CAPACITY_FACTOR), s)
    scores = jnp.einsum('bsd,d->bs', hidden_states, router_weight) + router_bias
    scores = jax.nn.sigmoid(scores)
    topk_scores, topk_indices, flat_idx, aux = _topk(scores)
    selected = _gather_rows(
        hidden_states.reshape(b * s, d), flat_idx
    ).reshape(b, k, d)
    return (selected, topk_indices, topk_scores, aux)
